# baseline (speedup 1.0000x reference)
import functools

import jax
import jax.numpy as jnp
from jax import lax
from jax.experimental import pallas as pl
from jax.experimental.pallas import tpu as pltpu
from jax.experimental.pallas import tpu_sc as plsc

EMBED_DIM = 512
TIMESTEPS = 1000
TBL = 1024
BATCH = 16384
NWORD = EMBED_DIM // 2

NC = 2
NS = 16
NW = NC * NS
BPW = BATCH // NW
CH = 64
NCHUNK = BPW // CH
NG = EMBED_DIM // 32


def _mlp_table_body(enc_ref, w1_ref, b1_ref, w2_ref, b2_ref, out_ref):
    h = jnp.dot(enc_ref[...], w1_ref[...], preferred_element_type=jnp.float32)
    h = h + b1_ref[...]
    h = jnp.where(h >= 0, h, 0.01 * h)
    o = jnp.dot(h, w2_ref[...], preferred_element_type=jnp.float32)
    ob = (o + b2_ref[...]).astype(jnp.bfloat16)
    o4 = ob.reshape(TIMESTEPS, NG, 2, 16)
    lo = lax.bitcast_convert_type(o4[:, :, 0, :], jnp.int16).astype(jnp.int32)
    hi = lax.bitcast_convert_type(o4[:, :, 1, :], jnp.int16).astype(jnp.int32)
    word = (lo & 0xFFFF) | (hi << 16)
    out_ref[pl.ds(0, TIMESTEPS), :] = word.reshape(TIMESTEPS, NWORD)


def _compute_table(encoding, W1, b1, W2, b2):
    return pl.pallas_call(
        _mlp_table_body,
        out_shape=jax.ShapeDtypeStruct((TBL, NWORD), jnp.int32),
    )(encoding, W1, b1.reshape(1, EMBED_DIM), W2, b2.reshape(1, EMBED_DIM))


def _gather_body(tab_hbm, idx_hbm, out_hbm,
                 idx_v, rb0, rb1, rf0, rf1, g0, g1, w0, w1):
    s = lax.axis_index("s")
    wid = s * NC + lax.axis_index("c")
    base = wid * BPW
    pltpu.sync_copy(idx_hbm.at[wid], idx_v)
    rbs = (rb0, rb1)
    rfs = (rf0, rf1)
    gsems = (g0, g1)
    wsems = (w0, w1)
    gh = [None, None]
    wh = [None, None]
    gh[0] = pltpu.async_copy(tab_hbm.at[idx_v.at[0]], rb0, g0)
    for j in range(NCHUNK):
        b = j % 2
        gh[b].wait()
        if j + 1 < NCHUNK:
            nb = 1 - b
            gh[nb] = pltpu.async_copy(tab_hbm.at[idx_v.at[j + 1]], rbs[nb], gsems[nb])
        if wh[b] is not None:
            wh[b].wait()
        rb = rbs[b]
        rf = rfs[b]

        @plsc.parallel_loop(0, CH, 1, unroll=2)
        def _cvt(r):
            for g in range(NG):
                v = rb[r, pl.ds(16 * g, 16)]
                a = plsc.bitcast(v << 16, jnp.float32)
                c = plsc.bitcast(v & jnp.int32(-65536), jnp.float32)
                rf[r, pl.ds(32 * g, 16)] = a
                rf[r, pl.ds(32 * g + 16, 16)] = c

        wh[b] = pltpu.async_copy(rf, out_hbm.at[pl.ds(base + j * CH, CH)], wsems[b])
    wh[0].wait()
    wh[1].wait()


_gather = functools.partial(
    pl.kernel,
    out_type=jax.ShapeDtypeStruct((BATCH, EMBED_DIM), jnp.float32),
    mesh=plsc.VectorSubcoreMesh(core_axis_name="c", subcore_axis_name="s"),
    compiler_params=pltpu.CompilerParams(needs_layout_passes=False),
    scratch_types=[
        pltpu.VMEM((NCHUNK, CH), jnp.int32),
        pltpu.VMEM((CH, NWORD), jnp.int32),
        pltpu.VMEM((CH, NWORD), jnp.int32),
        pltpu.VMEM((CH, EMBED_DIM), jnp.float32),
        pltpu.VMEM((CH, EMBED_DIM), jnp.float32),
        pltpu.SemaphoreType.DMA,
        pltpu.SemaphoreType.DMA,
        pltpu.SemaphoreType.DMA,
        pltpu.SemaphoreType.DMA,
    ],
)(_gather_body)


def kernel(t, encoding, W1, b1, W2, b2):
    table = _compute_table(encoding, W1, b1, W2, b2)
    idx = t.astype(jnp.int32).reshape(NW, NCHUNK, CH)
    return _gather(table, idx)


# speedup vs baseline: 1.5189x; 1.5189x over previous
import functools

import jax
import jax.numpy as jnp
from jax import lax
from jax.experimental import pallas as pl
from jax.experimental.pallas import tpu as pltpu
from jax.experimental.pallas import tpu_sc as plsc

EMBED_DIM = 512
TIMESTEPS = 1000
TBL = 1024
BATCH = 16384

NC = 2
NS = 16
NW = NC * NS
BPW = BATCH // NW
CH = 128
NCHUNK = BPW // CH


def _mlp_table_body(enc_ref, w1_ref, b1_ref, w2_ref, b2_ref, out_ref):
    h = jnp.dot(enc_ref[...], w1_ref[...], preferred_element_type=jnp.float32)
    h = h + b1_ref[...]
    h = jnp.where(h >= 0, h, 0.01 * h)
    o = jnp.dot(h, w2_ref[...], preferred_element_type=jnp.float32)
    out_ref[pl.ds(0, TIMESTEPS), :] = o + b2_ref[...]


def _compute_table(encoding, W1, b1, W2, b2):
    return pl.pallas_call(
        _mlp_table_body,
        out_shape=jax.ShapeDtypeStruct((TBL, EMBED_DIM), jnp.float32),
    )(encoding, W1, b1.reshape(1, EMBED_DIM), W2, b2.reshape(1, EMBED_DIM))


def _gather_body(table_hbm, idx_hbm, out_hbm, idx_v, rows, gsem, wsem):
    s = lax.axis_index("s")
    wid = s * NC + lax.axis_index("c")
    base = wid * BPW
    pltpu.sync_copy(idx_hbm.at[wid], idx_v)
    wh = None
    for j in range(NCHUNK):
        if wh is not None:
            wh.wait()
        pltpu.async_copy(table_hbm.at[idx_v.at[j]], rows, gsem).wait()
        wh = pltpu.async_copy(rows, out_hbm.at[pl.ds(base + j * CH, CH)], wsem)
    wh.wait()


_gather = functools.partial(
    pl.kernel,
    out_type=jax.ShapeDtypeStruct((BATCH, EMBED_DIM), jnp.float32),
    mesh=plsc.VectorSubcoreMesh(core_axis_name="c", subcore_axis_name="s"),
    scratch_types=[
        pltpu.VMEM((NCHUNK, CH), jnp.int32),
        pltpu.VMEM((CH, EMBED_DIM), jnp.float32),
        pltpu.SemaphoreType.DMA,
        pltpu.SemaphoreType.DMA,
    ],
)(_gather_body)


def kernel(t, encoding, W1, b1, W2, b2):
    table = _compute_table(encoding, W1, b1, W2, b2)
    idx = t.astype(jnp.int32).reshape(NW, NCHUNK, CH)
    return _gather(table, idx)
